# Initial kernel scaffold; baseline (speedup 1.0000x reference)
import jax
import jax.numpy as jnp
from jax.experimental import pallas as pl


def kernel(aatype, msa, msa_emb, seq_emb, relpos_emb, relpos_W, relpos_b):
    raise NotImplementedError("write your pallas kernel here")


# speedup vs baseline: 14.7286x; 14.7286x over previous
import jax
import jax.numpy as jnp
from jax.experimental import pallas as pl


def _onehot2(ids2d, k):
    a, b2 = ids2d.shape
    ids3 = jax.lax.broadcast_in_dim(ids2d, (a, b2, k), (0, 1))
    iota = jax.lax.broadcasted_iota(jnp.int32, (a, b2, k), 2)
    return (ids3 == iota).astype(jnp.float32).reshape(a * b2, k)


def _m_body(msa_ref, emb_ref, out_ref):
    idx = msa_ref[...]
    rb, l = idx.shape
    k, cm = emb_ref.shape
    oh = _onehot2(idx, k)
    rows = jnp.dot(oh, emb_ref[...], preferred_element_type=jnp.float32)
    out_ref[...] = rows.reshape(rb, l, cm)


def _z_body(afull_ref, ablk_ref, semb_ref, remb_ref, w_ref, b_ref, out_ref):
    l = afull_ref.shape[1]
    ib = ablk_ref.shape[2]
    na, ch = semb_ref.shape
    nr = remb_ref.shape[0]

    semb = semb_ref[...]
    s_full = jnp.dot(_onehot2(afull_ref[...], na), semb,
                     preferred_element_type=jnp.float32)
    s_blk = jnp.dot(_onehot2(ablk_ref[...].reshape(1, ib), na), semb,
                    preferred_element_type=jnp.float32)

    ptab = jnp.dot(remb_ref[...], w_ref[...],
                   preferred_element_type=jnp.float32) + b_ref[...]

    i0 = pl.program_id(0) * ib
    ivec = i0 + jax.lax.broadcasted_iota(jnp.int32, (ib, l), 0)
    jvec = jax.lax.broadcasted_iota(jnp.int32, (ib, l), 1)
    rel = jnp.clip(ivec - jvec, -32, 32) + 32
    pt = jnp.dot(_onehot2(rel, nr), ptab,
                 preferred_element_type=jnp.float32)

    zeros_i = jnp.zeros((ib, ch), jnp.float32)
    zeros_j = jnp.zeros((l, ch), jnp.float32)
    si = jnp.concatenate([s_blk, zeros_i], axis=-1)
    sj = jnp.concatenate([zeros_j, s_full], axis=-1)
    z = pt.reshape(ib, l, 2 * ch) + si[:, None, :] + sj[None, :, :]
    out_ref[...] = z.reshape(1, ib, l, 2 * ch)


def kernel(aatype, msa, msa_emb, seq_emb, relpos_emb, relpos_W, relpos_b):
    b, n, l = msa.shape
    k, cm = msa_emb.shape
    ch = seq_emb.shape[1]
    cz = 2 * ch

    msa2 = msa.reshape(b * n, l).astype(jnp.int32)
    aat2 = aatype.reshape(b, l).astype(jnp.int32)

    rb = 16
    m_flat = pl.pallas_call(
        _m_body,
        grid=((b * n) // rb,),
        in_specs=[
            pl.BlockSpec((rb, l), lambda i: (i, 0)),
            pl.BlockSpec((k, cm), lambda i: (0, 0)),
        ],
        out_specs=pl.BlockSpec((rb, l, cm), lambda i: (i, 0, 0)),
        out_shape=jax.ShapeDtypeStruct((b * n, l, cm), jnp.float32),
    )(msa2, msa_emb)
    m = m_flat.reshape(b, n, l, cm)

    ib = 32
    z = pl.pallas_call(
        _z_body,
        grid=(l // ib,),
        in_specs=[
            pl.BlockSpec((1, l), lambda i: (0, 0)),
            pl.BlockSpec((1, 1, ib), lambda i: (i, 0, 0)),
            pl.BlockSpec((k, ch), lambda i: (0, 0)),
            pl.BlockSpec((65, ch), lambda i: (0, 0)),
            pl.BlockSpec((ch, cz), lambda i: (0, 0)),
            pl.BlockSpec((1, cz), lambda i: (0, 0)),
        ],
        out_specs=pl.BlockSpec((1, ib, l, cz), lambda i: (0, i, 0, 0)),
        out_shape=jax.ShapeDtypeStruct((1, l, l, cz), jnp.float32),
    )(aat2, aat2.reshape(b * l // ib, 1, ib), seq_emb, relpos_emb, relpos_W,
      relpos_b.reshape(1, cz))
    z = jnp.broadcast_to(z, (b, l, l, cz))

    return (m, z)
